# baseline (speedup 1.0000x reference)
import functools

import jax
import jax.numpy as jnp
from jax import lax
from jax.experimental import pallas as pl
from jax.experimental.pallas import tpu as pltpu
from jax.experimental.pallas import tpu_sc as plsc

D = 64
NW = 32
G = 128
NBUF = 4
LA = 3


@functools.cache
def _make_gather(T, B):
    NBLK = B // G
    assert NBLK == NW and T % NBUF == 0 and T >= 2 * NBUF
    mesh = plsc.VectorSubcoreMesh(core_axis_name="c", subcore_axis_name="s")

    @functools.partial(
        pl.kernel,
        mesh=mesh,
        out_type=jax.ShapeDtypeStruct((T, D // 8, NBLK, 8, G), jnp.float32),
        scratch_types=[
            pltpu.VMEM((T, G), jnp.int32),
            pltpu.VMEM((NBUF, G, D), jnp.float32),
            pltpu.VMEM((2, D // 8, 8, G), jnp.float32),
            pltpu.SemaphoreType.DMA((NBUF,)),
            pltpu.SemaphoreType.DMA((2,)),
        ],
        compiler_params=pltpu.CompilerParams(
            use_tc_tiling_on_sc=False, needs_layout_passes=False),
    )
    def gather_k(idx_hbm, table_hbm, out_hbm, idx_v, rows_v, btile_v,
                 gsem, ssem):
        w = lax.axis_index("s") * 2 + lax.axis_index("c")
        pltpu.sync_copy(idx_hbm.at[:, pl.ds(w * G, G)], idx_v)

        def g_start(b, t):
            pltpu.make_async_copy(
                table_hbm.at[idx_v.at[t]], rows_v.at[b], gsem.at[b]).start()

        def g_wait(b):
            pltpu.make_async_copy(
                table_hbm.at[idx_v.at[0]], rows_v.at[b], gsem.at[b]).wait()

        def s_start(p, t):
            pltpu.make_async_copy(
                btile_v.at[p], out_hbm.at[t, :, w], ssem.at[p]).start()

        def s_wait(p):
            pltpu.make_async_copy(
                btile_v.at[p], out_hbm.at[0, :, w], ssem.at[p]).wait()

        iota = lax.iota(jnp.int32, 16)

        def transpose_tile(b, p):
            def jt_body(jt, carry):
                for jr in range(8):
                    col = jnp.full((16,), jt * 8 + jr, jnp.int32)
                    for m in range(8):
                        row = iota + (16 * m)
                        v = plsc.load_gather(rows_v.at[b], [row, col])
                        btile_v[p, jt, jr, pl.ds(16 * m, 16)] = v
                return carry

            lax.fori_loop(0, 8, jt_body, 0, unroll=False)

        for b in range(LA):
            g_start(b, b)

        def body(g, carry):
            for v in range(NBUF):
                t = g * NBUF + v
                p = v % 2
                g_wait(v)

                @pl.when(t + LA < T)
                def _():
                    g_start((v + LA) % NBUF, t + LA)

                @pl.when(t >= 2)
                def _():
                    s_wait(p)

                transpose_tile(v, p)
                s_start(p, t)
            return carry

        lax.fori_loop(0, T // NBUF, body, 0)

        for p in range(2):
            s_wait(p)

    return gather_k


def kernel(x, table):
    bsz, hist = x.shape
    xt = x.T
    out5 = _make_gather(hist, bsz)(xt, table)
    return out5.transpose(2, 4, 0, 1, 3).reshape(bsz, hist, D)


# speedup vs baseline: 1.4290x; 1.4290x over previous
import functools

import jax
import jax.numpy as jnp
from jax import lax
from jax.experimental import pallas as pl
from jax.experimental.pallas import tpu as pltpu
from jax.experimental.pallas import tpu_sc as plsc

D = 64
NW = 32
G = 128
NBUF = 4
LA = 3


@functools.cache
def _make_gather(T, B):
    NBLK = B // G
    assert NBLK == NW and T % NBUF == 0 and T >= 2 * NBUF
    mesh = plsc.VectorSubcoreMesh(core_axis_name="c", subcore_axis_name="s")

    @functools.partial(
        pl.kernel,
        mesh=mesh,
        out_type=jax.ShapeDtypeStruct((T, D // 8, NBLK, 8, G), jnp.float32),
        scratch_types=[
            pltpu.VMEM((T, G), jnp.int32),
            pltpu.VMEM((NBUF, G, D), jnp.float32),
            pltpu.VMEM((2, D // 8, 8, G), jnp.float32),
            pltpu.SemaphoreType.DMA((NBUF,)),
            pltpu.SemaphoreType.DMA((2,)),
        ],
        compiler_params=pltpu.CompilerParams(
            use_tc_tiling_on_sc=False, needs_layout_passes=False),
    )
    def gather_k(idx_hbm, table_hbm, out_hbm, idx_v, rows_v, btile_v,
                 gsem, ssem):
        w = lax.axis_index("s") * 2 + lax.axis_index("c")
        pltpu.sync_copy(idx_hbm.at[:, pl.ds(w * G, G)], idx_v)

        def g_start(b, t):
            pltpu.make_async_copy(
                table_hbm.at[idx_v.at[t]], rows_v.at[b], gsem.at[b]).start()

        def g_wait(b):
            pltpu.make_async_copy(
                table_hbm.at[idx_v.at[0]], rows_v.at[b], gsem.at[b]).wait()

        def s_start(p, t):
            pltpu.make_async_copy(
                btile_v.at[p], out_hbm.at[t, :, w], ssem.at[p]).start()

        def s_wait(p):
            pltpu.make_async_copy(
                btile_v.at[p], out_hbm.at[0, :, w], ssem.at[p]).wait()

        iota = lax.iota(jnp.int32, 16)

        def transpose_tile(b, p):
            @plsc.parallel_loop(0, 64, unroll=8)
            def _(q):
                jt = q >> 3
                jr = q & 7
                col = jnp.full((16,), jt * 8 + jr, jnp.int32)
                for m in range(8):
                    row = iota + (16 * m)
                    v = plsc.load_gather(rows_v.at[b], [row, col])
                    btile_v[p, jt, jr, pl.ds(16 * m, 16)] = v

        for b in range(LA):
            g_start(b, b)

        def body(g, carry):
            for v in range(NBUF):
                t = g * NBUF + v
                p = v % 2
                g_wait(v)

                @pl.when(t + LA < T)
                def _():
                    g_start((v + LA) % NBUF, t + LA)

                @pl.when(t >= 2)
                def _():
                    s_wait(p)

                transpose_tile(v, p)
                s_start(p, t)
            return carry

        lax.fori_loop(0, T // NBUF, body, 0)

        for p in range(2):
            s_wait(p)

    return gather_k


def kernel(x, table):
    bsz, hist = x.shape
    xt = x.T
    out5 = _make_gather(hist, bsz)(xt, table)
    return out5.transpose(2, 4, 0, 1, 3).reshape(bsz, hist, D)


# speedup vs baseline: 1.9134x; 1.3389x over previous
import functools

import jax
import jax.numpy as jnp
from jax import lax
from jax.experimental import pallas as pl
from jax.experimental.pallas import tpu as pltpu
from jax.experimental.pallas import tpu_sc as plsc

D = 64
DP = 128
NW = 32
G = 128
NBUF = 4
LA = 3
CT = 8192


def _tc_repack(tt):
    V = tt.shape[1]
    grid = (V + CT - 1) // CT

    def repack_body(tt_ref, out_ref):
        out_ref[:, 0:D] = tt_ref[...].T

    return pl.pallas_call(
        repack_body,
        grid=(grid,),
        in_specs=[pl.BlockSpec((D, CT), lambda c: (0, c))],
        out_specs=pl.BlockSpec((CT, DP), lambda c: (c, 0)),
        out_shape=jax.ShapeDtypeStruct((V, DP), jnp.float32),
    )(tt)


@functools.cache
def _make_gather(T, B):
    NBLK = B // G
    assert NBLK == NW and T % NBUF == 0 and T >= 2 * NBUF
    mesh = plsc.VectorSubcoreMesh(core_axis_name="c", subcore_axis_name="s")

    @functools.partial(
        pl.kernel,
        mesh=mesh,
        out_type=jax.ShapeDtypeStruct((T, D // 8, NBLK, 8, G), jnp.float32),
        scratch_types=[
            pltpu.VMEM((T, G), jnp.int32),
            pltpu.VMEM((NBUF, G, DP), jnp.float32),
            pltpu.VMEM((2, D // 8, 8, G), jnp.float32),
            pltpu.SemaphoreType.DMA((NBUF,)),
            pltpu.SemaphoreType.DMA((2,)),
        ],
        compiler_params=pltpu.CompilerParams(
            use_tc_tiling_on_sc=False, needs_layout_passes=False),
    )
    def gather_k(idx_hbm, table_hbm, out_hbm, idx_v, rows_v, btile_v,
                 gsem, ssem):
        w = lax.axis_index("s") * 2 + lax.axis_index("c")
        pltpu.sync_copy(idx_hbm.at[:, pl.ds(w * G, G)], idx_v)

        def g_start(b, t):
            pltpu.make_async_copy(
                table_hbm.at[idx_v.at[t]], rows_v.at[b], gsem.at[b]).start()

        def g_wait(b):
            pltpu.make_async_copy(
                table_hbm.at[idx_v.at[0]], rows_v.at[b], gsem.at[b]).wait()

        def s_start(p, t):
            pltpu.make_async_copy(
                btile_v.at[p], out_hbm.at[t, :, w], ssem.at[p]).start()

        def s_wait(p):
            pltpu.make_async_copy(
                btile_v.at[p], out_hbm.at[0, :, w], ssem.at[p]).wait()

        iota = lax.iota(jnp.int32, 16)

        def transpose_tile(b, p):
            @plsc.parallel_loop(0, 64, unroll=8)
            def _(q):
                jt = q >> 3
                jr = q & 7
                col = jnp.full((16,), jt * 8 + jr, jnp.int32)
                for m in range(8):
                    row = iota + (16 * m)
                    v = plsc.load_gather(rows_v.at[b], [row, col])
                    btile_v[p, jt, jr, pl.ds(16 * m, 16)] = v

        def visit(t, b, p, do_swait, do_gstart):
            g_wait(b)
            if do_gstart:
                g_start((b + LA) % NBUF, t + LA)
            if do_swait:
                s_wait(p)
            transpose_tile(b, p)
            s_start(p, t)

        for b in range(LA):
            g_start(b, b)

        def body(g, carry):
            for v in range(NBUF):
                t = g * NBUF + v
                p = v % 2

                g_wait(v)

                @pl.when(t + LA < T)
                def _():
                    g_start((v + LA) % NBUF, t + LA)

                @pl.when(t >= 2)
                def _():
                    s_wait(p)

                transpose_tile(v, p)
                s_start(p, t)
            return carry

        lax.fori_loop(0, T // NBUF, body, 0)

        for p in range(2):
            s_wait(p)

    return gather_k


def kernel(x, table):
    bsz, hist = x.shape
    xt = x.T
    table_rm = _tc_repack(table.T)
    out5 = _make_gather(hist, bsz)(xt, table_rm)
    return out5.transpose(2, 4, 0, 1, 3).reshape(bsz, hist, D)
